# baseline (speedup 1.0000x reference)
import functools

import jax
import jax.numpy as jnp
from jax import lax
from jax.experimental import pallas as pl
from jax.experimental.pallas import tpu as pltpu
from jax.experimental.pallas import tpu_sc as plsc

R = 1024
C = 100000
K = 200
NTOT = R * K

_info = plsc.get_sparse_core_info()
NC, NS, L = _info.num_cores, _info.num_subcores, _info.num_lanes
NW = NC * NS
PER_W = NTOT // NW
CHUNK = 128
NCHUNK = PER_W // CHUNK
VPC = CHUNK // L


def _sc_gather(x_flat, idx_flat):
    mesh = plsc.VectorSubcoreMesh(core_axis_name="c", subcore_axis_name="s")

    @functools.partial(
        pl.kernel,
        mesh=mesh,
        out_type=jax.ShapeDtypeStruct((NTOT,), jnp.float32),
        scratch_types=[
            pltpu.VMEM((PER_W,), jnp.int32),
            pltpu.VMEM((PER_W,), jnp.float32),
            pltpu.SemaphoreType.DMA,
        ],
    )
    def body(x_hbm, idx_hbm, out_hbm, idx_v, val_v, sem):
        w = lax.axis_index("s") * NC + lax.axis_index("c")
        base = w * PER_W
        pltpu.sync_copy(idx_hbm.at[pl.ds(base, PER_W)], idx_v)

        def chunk_body(j, carry):
            lanes = lax.iota(jnp.int32, L)
            for v in range(VPC):
                o = j * CHUNK + v * L
                t = base + o
                q = t // K
                rem = t - q * K
                row = q + jnp.where(rem + lanes >= K, 1, 0)
                idx_v[pl.ds(o, L)] = idx_v[pl.ds(o, L)] + row * C
            pltpu.async_copy(
                x_hbm.at[idx_v.at[pl.ds(j * CHUNK, CHUNK)]],
                val_v.at[pl.ds(j * CHUNK, CHUNK)],
                sem,
            )
            return carry

        lax.fori_loop(0, NCHUNK, chunk_body, 0)
        pltpu.make_async_copy(x_hbm.at[pl.ds(0, PER_W)], val_v, sem).wait()
        pltpu.sync_copy(val_v, out_hbm.at[pl.ds(base, PER_W)])

    return body(x_flat, idx_flat)


def kernel(x, idx):
    x_flat = x.reshape(R * C)
    idx_flat = idx.reshape(NTOT).astype(jnp.int32)
    return _sc_gather(x_flat, idx_flat).reshape(R, K)


# speedup vs baseline: 26.6017x; 26.6017x over previous
import functools

import jax
import jax.numpy as jnp
from jax import lax
from jax.experimental import pallas as pl
from jax.experimental.pallas import tpu as pltpu
from jax.experimental.pallas import tpu_sc as plsc

R = 1024
C = 100000
K = 200
NTOT = R * K
NX = R * C

_info = plsc.get_sparse_core_info()
NC, NS, L = _info.num_cores, _info.num_subcores, _info.num_lanes
NW = NC * NS
PER_W = NTOT // NW
CHUNK = 128
NCHUNK = PER_W // CHUNK
VPC = CHUNK // L


def _sc_gather(x_phys, idx_flat):
    mesh = plsc.VectorSubcoreMesh(core_axis_name="c", subcore_axis_name="s")

    @functools.partial(
        pl.kernel,
        mesh=mesh,
        out_type=jax.ShapeDtypeStruct((NTOT,), jnp.float32),
        scratch_types=[
            pltpu.VMEM((PER_W,), jnp.int32),
            pltpu.VMEM((PER_W,), jnp.float32),
            pltpu.SemaphoreType.DMA,
        ],
    )
    def body(x_hbm, idx_hbm, out_hbm, idx_v, val_v, sem):
        w = lax.axis_index("s") * NC + lax.axis_index("c")
        base = w * PER_W
        pltpu.sync_copy(idx_hbm.at[pl.ds(base, PER_W)], idx_v)

        def chunk_body(j, carry):
            lanes = lax.iota(jnp.int32, L)
            for v in range(VPC):
                o = j * CHUNK + v * L
                t = base + o
                q = t // K
                rem = t - q * K
                r = q + jnp.where(rem + lanes >= K, 1, 0)
                c = idx_v[pl.ds(o, L)]
                addr = (((c & ~7) << 10) + ((r >> 7) << 10)
                        + ((c & 7) << 7) + (r & 127))
                idx_v[pl.ds(o, L)] = addr
            pltpu.async_copy(
                x_hbm.at[idx_v.at[pl.ds(j * CHUNK, CHUNK)]],
                val_v.at[pl.ds(j * CHUNK, CHUNK)],
                sem,
            )
            return carry

        lax.fori_loop(0, NCHUNK, chunk_body, 0)
        pltpu.make_async_copy(x_hbm.at[pl.ds(0, PER_W)], val_v, sem).wait()
        pltpu.sync_copy(val_v, out_hbm.at[pl.ds(base, PER_W)])

    return body(x_phys, idx_flat)


def kernel(x, idx):
    x_phys = (
        x.reshape(8, 128, C // 8, 8).transpose(2, 0, 3, 1).reshape(NX)
    )
    idx_flat = idx.reshape(NTOT).astype(jnp.int32)
    return _sc_gather(x_phys, idx_flat).reshape(R, K)


# speedup vs baseline: 31.2674x; 1.1754x over previous
import functools

import jax
import jax.numpy as jnp
from jax import lax
from jax.experimental import pallas as pl
from jax.experimental.pallas import tpu as pltpu
from jax.experimental.pallas import tpu_sc as plsc

R = 1024
C = 100000
K = 200
NTOT = R * K
NX = R * C

_info = plsc.get_sparse_core_info()
NC, NS, L = _info.num_cores, _info.num_subcores, _info.num_lanes
NW = NC * NS
PER_W = NTOT // NW
CHUNK = 128
NCHUNK = PER_W // CHUNK
VPC = CHUNK // L


def _sc_gather(x_phys, idx_phys):
    mesh = plsc.VectorSubcoreMesh(core_axis_name="c", subcore_axis_name="s")

    @functools.partial(
        pl.kernel,
        mesh=mesh,
        out_type=jax.ShapeDtypeStruct((NTOT,), jnp.float32),
        scratch_types=[
            pltpu.VMEM((PER_W,), jnp.int32),
            pltpu.VMEM((PER_W,), jnp.float32),
            pltpu.SemaphoreType.DMA,
        ],
    )
    def body(x_hbm, idx_hbm, out_hbm, idx_v, val_v, sem):
        w = lax.axis_index("s") * NC + lax.axis_index("c")
        base = w * PER_W
        pltpu.sync_copy(idx_hbm.at[pl.ds(base, PER_W)], idx_v)

        def chunk_body(j, carry):
            lanes = lax.iota(jnp.int32, L)
            for v in range(VPC):
                o = j * CHUNK + v * L
                k = base + o
                tc = (k >> 10) & 7
                lbase = k & 127
                rlo = lbase + lanes
                c = idx_v[pl.ds(o, L)]
                addr = (((c & ~7) << 10) + ((c & 7) << 7)
                        + ((tc << 10) + rlo))
                idx_v[pl.ds(o, L)] = addr
            pltpu.async_copy(
                x_hbm.at[idx_v.at[pl.ds(j * CHUNK, CHUNK)]],
                val_v.at[pl.ds(j * CHUNK, CHUNK)],
                sem,
            )
            return carry

        lax.fori_loop(0, NCHUNK, chunk_body, 0)
        pltpu.make_async_copy(x_hbm.at[pl.ds(0, PER_W)], val_v, sem).wait()
        pltpu.sync_copy(val_v, out_hbm.at[pl.ds(base, PER_W)])

    return body(x_phys, idx_phys)


def kernel(x, idx):
    x_phys = x.reshape(8, 128, C // 8, 8).transpose(2, 0, 3, 1).reshape(NX)
    idx_phys = (
        idx.astype(jnp.int32)
        .reshape(8, 128, K // 8, 8)
        .transpose(2, 0, 3, 1)
        .reshape(NTOT)
    )
    out_phys = _sc_gather(x_phys, idx_phys)
    return (
        out_phys.reshape(K // 8, 8, 8, 128).transpose(1, 3, 0, 2).reshape(R, K)
    )
